# baseline (speedup 1.0000x reference)
import functools

import jax
import jax.numpy as jnp
from jax.experimental import pallas as pl
from jax.experimental.pallas import tpu as pltpu

_ALPHA = (0.12, 0.31, 0.44, 0.27, 0.53, 0.19, 0.66, 0.38,
          0.22, 0.49, 0.17, 0.61, 0.34, 0.28, 0.57, 0.41)
_GAMMA = 2.0

_TR = 8
_IMGS = 8


def _tree(fn, xs):
    xs = list(xs)
    while len(xs) > 1:
        nxt = [fn(xs[i], xs[i + 1]) for i in range(0, len(xs) - 1, 2)]
        if len(xs) % 2:
            nxt.append(xs[-1])
        xs = nxt
    return xs[0]


def _bit_select(vals, t, nbits):
    cur = list(vals)
    for k in range(nbits):
        bit = (t & (1 << k)) != 0
        cur = [jnp.where(bit, cur[2 * i + 1], cur[2 * i])
               for i in range(len(cur) // 2)]
    return cur[0]


def _focal_kernel(xa_ref, xb_ref, ta_ref, tb_ref, out_ref, acc_ref, *, C, G,
                  rows, alpha, nbits, steps, inv_m):
    i = pl.program_id(0)
    acc = None
    for g in range(G):
        x_ref, t_ref = (xa_ref, ta_ref) if g < G // 2 else (xb_ref, tb_ref)
        gg = g % (G // 2)
        for r0 in range(0, rows, _TR):
            sl = slice(r0, r0 + _TR)
            xs = [x_ref[gg, c, sl, :] for c in range(C)]
            t = t_ref[gg, 0, sl, :]

            m = _tree(jnp.maximum, xs)
            se = _tree(lambda a, b: a + b, [jnp.exp(x - m) for x in xs])
            lse = jnp.log(se) + m

            xt = _bit_select(xs, t, nbits)
            at = _bit_select([jnp.float32(a) for a in alpha], t, nbits)

            logpt = xt - lse
            pt = jnp.exp(logpt)
            omp = jnp.maximum(1.0 - pt, 0.0)
            contrib = (omp * omp) * (at * logpt)
            acc = contrib if acc is None else acc + contrib

    @pl.when(i == 0)
    def _():
        acc_ref[...] = acc

    @pl.when(i != 0)
    def _():
        acc_ref[...] += acc

    @pl.when(i == steps - 1)
    def _():
        out_ref[0, 0] = jnp.sum(acc_ref[...]) * jnp.float32(-inv_m)


def kernel(logits, target):
    N, C = logits.shape[0], logits.shape[1]
    HW = 1
    for d in logits.shape[2:]:
        HW *= d
    M = N * HW
    assert HW % 128 == 0, "token count must be lane aligned"
    R = HW // 128

    x = logits.reshape(N, C, R, 128)
    t = target.reshape(N, 1, R, 128)

    G = _IMGS if N % _IMGS == 0 else 2
    assert N % G == 0 and G % 2 == 0, "need an even image-group size"
    steps = N // G
    nbits = max(1, (C - 1).bit_length())
    assert C == len(_ALPHA) and (1 << nbits) == C
    assert R % _TR == 0

    kern = functools.partial(_focal_kernel, C=C, G=G, rows=R, alpha=_ALPHA,
                             nbits=nbits, steps=steps, inv_m=1.0 / M)
    total = pl.pallas_call(
        kern,
        out_shape=jax.ShapeDtypeStruct((1, 1), jnp.float32),
        grid=(steps,),
        in_specs=[
            pl.BlockSpec((G // 2, C, R, 128), lambda i: (2 * i, 0, 0, 0)),
            pl.BlockSpec((G // 2, C, R, 128), lambda i: (2 * i + 1, 0, 0, 0)),
            pl.BlockSpec((G // 2, 1, R, 128), lambda i: (2 * i, 0, 0, 0)),
            pl.BlockSpec((G // 2, 1, R, 128), lambda i: (2 * i + 1, 0, 0, 0)),
        ],
        out_specs=pl.BlockSpec(memory_space=pltpu.SMEM),
        scratch_shapes=[pltpu.VMEM((8, 128), jnp.float32)],
        compiler_params=pltpu.CompilerParams(
            dimension_semantics=("arbitrary",),
            vmem_limit_bytes=48 * 1024 * 1024),
    )(x, x, t, t)
    return total.reshape(())


# speedup vs baseline: 1.0166x; 1.0166x over previous
import functools

import jax
import jax.numpy as jnp
from jax.experimental import pallas as pl
from jax.experimental.pallas import tpu as pltpu

_ALPHA = (0.12, 0.31, 0.44, 0.27, 0.53, 0.19, 0.66, 0.38,
          0.22, 0.49, 0.17, 0.61, 0.34, 0.28, 0.57, 0.41)
_GAMMA = 2.0

_TR = 8
_IMGS = 8


def _tree(fn, xs):
    xs = list(xs)
    while len(xs) > 1:
        nxt = [fn(xs[i], xs[i + 1]) for i in range(0, len(xs) - 1, 2)]
        if len(xs) % 2:
            nxt.append(xs[-1])
        xs = nxt
    return xs[0]


def _bit_select(vals, t, nbits):
    cur = list(vals)
    for k in range(nbits):
        bit = (t & (1 << k)) != 0
        cur = [jnp.where(bit, cur[2 * i + 1], cur[2 * i])
               for i in range(len(cur) // 2)]
    return cur[0]


def _focal_kernel(x_ref, t_ref, out_ref, acc_ref, *, C, G, rows, alpha, nbits,
                  steps, inv_m):
    i = pl.program_id(0)
    acc = None
    for g in range(G):
        for r0 in range(0, rows, _TR):
            sl = slice(r0, r0 + _TR)
            xs = [x_ref[g, c, sl, :] for c in range(C)]
            t = t_ref[g, 0, sl, :]

            m = _tree(jnp.maximum, xs)
            ds = [x - m for x in xs]
            se = _tree(lambda a, b: a + b, [jnp.exp(d) for d in ds])

            dt = _bit_select(ds, t, nbits)
            at = _bit_select([jnp.float32(a) for a in alpha], t, nbits)

            logpt = dt - jnp.log(se)
            pt = jnp.exp(logpt)
            omp = 1.0 - pt
            contrib = (omp * omp) * (at * logpt)
            acc = contrib if acc is None else acc + contrib

    @pl.when(i == 0)
    def _():
        acc_ref[...] = acc

    @pl.when(i != 0)
    def _():
        acc_ref[...] += acc

    @pl.when(i == steps - 1)
    def _():
        out_ref[0, 0] = jnp.sum(acc_ref[...]) * jnp.float32(-inv_m)


def kernel(logits, target):
    N, C = logits.shape[0], logits.shape[1]
    HW = 1
    for d in logits.shape[2:]:
        HW *= d
    M = N * HW
    assert HW % 128 == 0, "token count must be lane aligned"
    R = HW // 128

    x = logits.reshape(N, C, R, 128)
    t = target.reshape(N, 1, R, 128)

    G = _IMGS if N % _IMGS == 0 else 2
    assert N % G == 0 and G % 2 == 0, "need an even image-group size"
    steps = N // G
    nbits = max(1, (C - 1).bit_length())
    assert C == len(_ALPHA) and (1 << nbits) == C
    assert R % _TR == 0

    kern = functools.partial(_focal_kernel, C=C, G=G, rows=R, alpha=_ALPHA,
                             nbits=nbits, steps=steps, inv_m=1.0 / M)
    total = pl.pallas_call(
        kern,
        out_shape=jax.ShapeDtypeStruct((1, 1), jnp.float32),
        grid=(steps,),
        in_specs=[
            pl.BlockSpec((G, C, R, 128), lambda i: (i, 0, 0, 0)),
            pl.BlockSpec((G, 1, R, 128), lambda i: (i, 0, 0, 0)),
        ],
        out_specs=pl.BlockSpec(memory_space=pltpu.SMEM),
        scratch_shapes=[pltpu.VMEM((8, 128), jnp.float32)],
        compiler_params=pltpu.CompilerParams(
            dimension_semantics=("arbitrary",),
            vmem_limit_bytes=48 * 1024 * 1024),
    )(x, t)
    return total.reshape(())


# speedup vs baseline: 1.1675x; 1.1485x over previous
import functools

import jax
import jax.numpy as jnp
from jax.experimental import pallas as pl
from jax.experimental.pallas import tpu as pltpu

_ALPHA = (0.12, 0.31, 0.44, 0.27, 0.53, 0.19, 0.66, 0.38,
          0.22, 0.49, 0.17, 0.61, 0.34, 0.28, 0.57, 0.41)
_GAMMA = 2.0

_TR = 8
_GT = 4
_DEPTH = 3


def _tree(fn, xs):
    xs = list(xs)
    while len(xs) > 1:
        nxt = [fn(xs[i], xs[i + 1]) for i in range(0, len(xs) - 1, 2)]
        if len(xs) % 2:
            nxt.append(xs[-1])
        xs = nxt
    return xs[0]


def _bit_select(vals, t, nbits):
    cur = list(vals)
    for k in range(nbits):
        bit = (t & (1 << k)) != 0
        cur = [jnp.where(bit, cur[2 * i + 1], cur[2 * i])
               for i in range(len(cur) // 2)]
    return cur[0]


def _tile_loss(xbuf, tbuf, s, C, rows, alpha, nbits):
    acc = None
    for g in range(_GT):
        for r0 in range(0, rows, _TR):
            sl = slice(r0, r0 + _TR)
            xs = [xbuf[s, g, c, sl, :] for c in range(C)]
            t = tbuf[s, g, 0, sl, :]

            m = _tree(jnp.maximum, xs)
            ds = [x - m for x in xs]
            se = _tree(lambda a, b: a + b, [jnp.exp(d) for d in ds])

            dt = _bit_select(ds, t, nbits)
            at = _bit_select([jnp.float32(a) for a in alpha], t, nbits)

            logpt = dt - jnp.log(se)
            pt = jnp.exp(logpt)
            omp = 1.0 - pt
            contrib = (omp * omp) * (at * logpt)
            acc = contrib if acc is None else acc + contrib
    return acc


def _focal_kernel(x_hbm, t_hbm, out_ref, xbuf, tbuf, xsem, tsem, *, C, rows,
                  tiles, alpha, nbits, inv_m):
    def start(k, s):
        pltpu.make_async_copy(
            x_hbm.at[pl.ds(k * _GT, _GT)], xbuf.at[s], xsem.at[s]).start()
        pltpu.make_async_copy(
            t_hbm.at[pl.ds(k * _GT, _GT)], tbuf.at[s], tsem.at[s]).start()

    for k in range(min(_DEPTH, tiles)):
        start(k, k % _DEPTH)

    def body(k, total):
        s = jax.lax.rem(k, _DEPTH)
        pltpu.make_async_copy(xbuf.at[s], xbuf.at[s], xsem.at[s]).wait()
        pltpu.make_async_copy(tbuf.at[s], tbuf.at[s], tsem.at[s]).wait()
        acc = _tile_loss(xbuf, tbuf, s, C, rows, alpha, nbits)

        @pl.when(k + _DEPTH < tiles)
        def _():
            start(k + _DEPTH, s)

        return total + acc

    total = jax.lax.fori_loop(0, tiles, body, jnp.zeros((_TR, 128), jnp.float32))
    out_ref[0, 0] = jnp.sum(total) * jnp.float32(-inv_m)


def kernel(logits, target):
    N, C = logits.shape[0], logits.shape[1]
    HW = 1
    for d in logits.shape[2:]:
        HW *= d
    M = N * HW
    assert HW % 128 == 0, "token count must be lane aligned"
    R = HW // 128

    x = logits.reshape(N, C, R, 128)
    t = target.reshape(N, 1, R, 128)

    assert N % _GT == 0, "image count must tile evenly"
    tiles = N // _GT
    nbits = max(1, (C - 1).bit_length())
    assert C == len(_ALPHA) and (1 << nbits) == C
    assert R % _TR == 0

    kern = functools.partial(_focal_kernel, C=C, rows=R, tiles=tiles,
                             alpha=_ALPHA, nbits=nbits, inv_m=1.0 / M)
    total = pl.pallas_call(
        kern,
        out_shape=jax.ShapeDtypeStruct((1, 1), jnp.float32),
        in_specs=[
            pl.BlockSpec(memory_space=pl.ANY),
            pl.BlockSpec(memory_space=pl.ANY),
        ],
        out_specs=pl.BlockSpec(memory_space=pltpu.SMEM),
        scratch_shapes=[
            pltpu.VMEM((_DEPTH, _GT, C, R, 128), jnp.float32),
            pltpu.VMEM((_DEPTH, _GT, 1, R, 128), jnp.int32),
            pltpu.SemaphoreType.DMA((_DEPTH,)),
            pltpu.SemaphoreType.DMA((_DEPTH,)),
        ],
        compiler_params=pltpu.CompilerParams(
            vmem_limit_bytes=48 * 1024 * 1024),
    )(x, t)
    return total.reshape(())
